# baseline (speedup 1.0000x reference)
import functools

import jax
import jax.numpy as jnp
from jax import lax
from jax.experimental import pallas as pl
from jax.experimental.pallas import tpu as pltpu
from jax.experimental.pallas import tpu_sc as plsc

VOCAB = 1000000
EMBED = 64
HIDDEN = 128
NUM_CLASS = 20
BATCH = 4096
HIST = 50

NC = 2
NS = 16
LANES = 16
NW = NC * NS
BAGS_PER_W = BATCH // NW
BAGS_PER_DMA = 2
ROWS_PER_DMA = BAGS_PER_DMA * HIST
CHUNKS = BAGS_PER_W // BAGS_PER_DMA


def _embed_bag_body(text_hbm, table_hbm, out_hbm, idx_v, rows_v, out_v, sem):
    wid = lax.axis_index("s") * NC + lax.axis_index("c")
    pltpu.sync_copy(text_hbm.at[wid], idx_v)

    def chunk_body(j, _):
        pltpu.async_copy(table_hbm.at[idx_v.at[j]], rows_v, sem).wait()
        for bag in range(BAGS_PER_DMA):
            zero = jnp.zeros((LANES,), jnp.float32)

            def row_body(r, acc):
                base = bag * HIST + r
                return tuple(
                    acc[c] + rows_v[base, pl.ds(c * LANES, LANES)]
                    for c in range(EMBED // LANES)
                )

            acc = lax.fori_loop(0, HIST, row_body, (zero,) * (EMBED // LANES))
            for c in range(EMBED // LANES):
                out_v[j * BAGS_PER_DMA + bag, pl.ds(c * LANES, LANES)] = acc[c]
        return 0

    lax.fori_loop(0, CHUNKS, chunk_body, 0)
    pltpu.sync_copy(out_v, out_hbm.at[pl.ds(wid * BAGS_PER_W, BAGS_PER_W)])


_embed_bag = pl.kernel(
    _embed_bag_body,
    out_type=jax.ShapeDtypeStruct((BATCH, EMBED), jnp.float32),
    mesh=plsc.VectorSubcoreMesh(core_axis_name="c", subcore_axis_name="s"),
    scratch_types=[
        pltpu.VMEM((CHUNKS, ROWS_PER_DMA), jnp.int32),
        pltpu.VMEM((ROWS_PER_DMA, EMBED), jnp.float32),
        pltpu.VMEM((BAGS_PER_W, EMBED), jnp.float32),
        pltpu.SemaphoreType.DMA,
    ],
    compiler_params=pltpu.CompilerParams(use_tc_tiling_on_sc=False),
)


def _mlp_body(x_ref, w1_ref, b1_ref, w2_ref, b2_ref, o_ref):
    x = x_ref[...] * (1.0 / HIST)
    h = jnp.tanh(
        lax.dot_general(x, w1_ref[...], (((1,), (1,)), ((), ())),
                        preferred_element_type=jnp.float32)
        + b1_ref[...]
    )
    logits = (
        lax.dot_general(h, w2_ref[...], (((1,), (1,)), ((), ())),
                        preferred_element_type=jnp.float32)
        + b2_ref[...]
    )
    m = jnp.max(logits, axis=-1, keepdims=True)
    e = jnp.exp(logits - m)
    o_ref[...] = e / jnp.sum(e, axis=-1, keepdims=True)


@jax.jit
def kernel(text, table, W1, b1, W2, b2):
    idx = text.astype(jnp.int32).reshape(NW, CHUNKS, ROWS_PER_DMA)
    sums = _embed_bag(idx, table)
    return pl.pallas_call(
        _mlp_body,
        out_shape=jax.ShapeDtypeStruct((BATCH, NUM_CLASS), jnp.float32),
    )(sums, W1, b1.reshape(1, HIDDEN), W2, b2.reshape(1, NUM_CLASS))
